# Initial kernel scaffold; baseline (speedup 1.0000x reference)
import jax
import jax.numpy as jnp
from jax.experimental import pallas as pl


def kernel(x, edges, eps0, W1_0, b1_0, W2_0, b2_0, eps1, W1_1, b1_1, W2_1, b2_1, eps2, W1_2, b1_2, W2_2, b2_2):
    raise NotImplementedError("write your pallas kernel here")


# speedup vs baseline: 4.4091x; 4.4091x over previous
import functools

import jax
import jax.numpy as jnp
from jax import lax
from jax.experimental import pallas as pl
from jax.experimental.pallas import tpu as pltpu
from jax.experimental.pallas import tpu_sc as plsc

N_NODES = 10000
N_EDGES = 320000
D = 128
D_OUT = 40

NC = 2
NS = 16
NW = NC * NS
E_PER_TILE = N_EDGES // NW
CHUNK = 80
N_ITERS = E_PER_TILE // CHUNK
ROWS_PER_TILE = 624
ROWS_REM = N_NODES - NS * ROWS_PER_TILE
REM_BASE = NS * ROWS_PER_TILE

_sc_mesh = plsc.VectorSubcoreMesh(core_axis_name="c", subcore_axis_name="s")


@functools.partial(
    pl.kernel,
    out_type=jax.ShapeDtypeStruct((NC * N_NODES, D), jnp.float32),
    mesh=_sc_mesh,
    scratch_types=[
        pltpu.VMEM((CHUNK,), jnp.int32),
        pltpu.VMEM((CHUNK,), jnp.int32),
        pltpu.VMEM((CHUNK, D), jnp.float32),
        pltpu.VMEM_SHARED((N_NODES, D), jnp.float32),
        pltpu.SemaphoreType.DMA,
    ],
)
def _sc_aggregate(h_hbm, src_hbm, dst_hbm, zeros_hbm, out_hbm,
                  src_v, dst_v, rows_v, acc_sh, sem):
    cid = lax.axis_index("c")
    sid = lax.axis_index("s")
    wid = sid * NC + cid

    row0 = sid * ROWS_PER_TILE
    pltpu.sync_copy(zeros_hbm, acc_sh.at[pl.ds(row0, ROWS_PER_TILE)])

    @pl.when(sid == 0)
    def _zero_rem():
        pltpu.sync_copy(zeros_hbm.at[pl.ds(0, ROWS_REM)],
                        acc_sh.at[pl.ds(REM_BASE, ROWS_REM)])

    plsc.subcore_barrier()

    def body(i, carry):
        base = wid * E_PER_TILE + i * CHUNK
        pltpu.sync_copy(src_hbm.at[pl.ds(base, CHUNK)], src_v)
        pltpu.sync_copy(dst_hbm.at[pl.ds(base, CHUNK)], dst_v)
        pltpu.async_copy(h_hbm.at[src_v], rows_v, sem).wait()
        pltpu.sync_copy(rows_v, acc_sh.at[dst_v], add=True)
        return carry

    lax.fori_loop(0, N_ITERS, body, 0)
    plsc.subcore_barrier()

    pltpu.sync_copy(acc_sh.at[pl.ds(row0, ROWS_PER_TILE)],
                    out_hbm.at[pl.ds(cid * N_NODES + row0, ROWS_PER_TILE)])

    @pl.when(sid == 0)
    def _out_rem():
        pltpu.sync_copy(acc_sh.at[pl.ds(REM_BASE, ROWS_REM)],
                        out_hbm.at[pl.ds(cid * N_NODES + REM_BASE, ROWS_REM)])


def _tc_mlp(x, a0, a1, eps, W1, b1, W2, b2, act):
    B = 1000

    def body(eps_ref, x_ref, a0_ref, a1_ref, w1_ref, b1_ref, w2_ref, b2_ref,
             o_ref):
        h = (1.0 + eps_ref[0]) * x_ref[...] + a0_ref[...] + a1_ref[...]
        h = jnp.dot(h, w1_ref[...], preferred_element_type=jnp.float32)
        h = jnp.maximum(h + b1_ref[...], 0.0)
        o = jnp.dot(h, w2_ref[...], preferred_element_type=jnp.float32)
        o = o + b2_ref[...]
        if act == "relu":
            o = jnp.maximum(o, 0.0)
        elif act == "log_softmax":
            col = lax.broadcasted_iota(jnp.int32, o.shape, 1)
            valid = col < D_OUT
            om = jnp.where(valid, o, -jnp.inf)
            m = jnp.max(om, axis=1, keepdims=True)
            e = jnp.where(valid, jnp.exp(om - m), 0.0)
            o = o - (m + jnp.log(jnp.sum(e, axis=1, keepdims=True)))
        o_ref[...] = o

    return pl.pallas_call(
        body,
        grid=(N_NODES // B,),
        in_specs=[
            pl.BlockSpec(memory_space=pltpu.SMEM),
            pl.BlockSpec((B, D), lambda i: (i, 0)),
            pl.BlockSpec((B, D), lambda i: (i, 0)),
            pl.BlockSpec((B, D), lambda i: (i, 0)),
            pl.BlockSpec((D, D), lambda i: (0, 0)),
            pl.BlockSpec((1, D), lambda i: (0, 0)),
            pl.BlockSpec((D, D), lambda i: (0, 0)),
            pl.BlockSpec((1, D), lambda i: (0, 0)),
        ],
        out_specs=pl.BlockSpec((B, D), lambda i: (i, 0)),
        out_shape=jax.ShapeDtypeStruct((N_NODES, D), jnp.float32),
    )(eps, x, a0, a1, W1, b1, W2, b2)


def kernel(x, edges, eps0, W1_0, b1_0, W2_0, b2_0, eps1, W1_1, b1_1, W2_1,
           b2_1, eps2, W1_2, b1_2, W2_2, b2_2):
    src = edges[0].astype(jnp.int32)
    dst = edges[1].astype(jnp.int32)
    zeros = jnp.zeros((ROWS_PER_TILE, D), jnp.float32)

    W2_2p = jnp.zeros((D, D), jnp.float32).at[:, :D_OUT].set(W2_2)
    b2_2p = jnp.zeros((D,), jnp.float32).at[:D_OUT].set(b2_2)

    def agg(h):
        parts = _sc_aggregate(h, src, dst, zeros)
        return parts[:N_NODES], parts[N_NODES:]

    layers = [
        (eps0, W1_0, b1_0, W2_0, b2_0, "relu"),
        (eps1, W1_1, b1_1, W2_1, b2_1, "relu"),
        (eps2, W1_2, b1_2, W2_2p, b2_2p, "log_softmax"),
    ]
    h = x
    for eps, W1, b1, W2, b2, act in layers:
        a0, a1 = agg(h)
        h = _tc_mlp(h, a0, a1, eps.reshape(1), W1, b1.reshape(1, D), W2,
                    b2.reshape(1, D), act)
    return h[:, :D_OUT]
